# baseline (speedup 1.0000x reference)
import jax
import jax.numpy as jnp
from jax.experimental import pallas as pl


def _mm_kernel(x_ref, y_ref, o_ref):
    o_ref[0] = jnp.dot(x_ref[0], y_ref[0], preferred_element_type=jnp.float32)


def kernel(Y, x):
    B, N, _ = Y.shape
    _, C, F, _ = x.shape
    M = C * F
    x2 = x.reshape(B, M, N)
    TN = 256
    out = pl.pallas_call(
        _mm_kernel,
        grid=(B, N // TN),
        in_specs=[
            pl.BlockSpec((1, M, N), lambda b, j: (b, 0, 0)),
            pl.BlockSpec((1, N, TN), lambda b, j: (b, 0, j)),
        ],
        out_specs=pl.BlockSpec((1, M, TN), lambda b, j: (b, 0, j)),
        out_shape=jax.ShapeDtypeStruct((B, M, N), jnp.float32),
    )(x2, Y)
    return out.reshape(B, C, F, N)


# speedup vs baseline: 1.3627x; 1.3627x over previous
import jax
import jax.numpy as jnp
from jax.experimental import pallas as pl


def _mm_kernel(x_ref, y_ref, o_ref):
    o_ref[0] = jnp.dot(x_ref[0], y_ref[0], preferred_element_type=jnp.float32)


def kernel(Y, x):
    B, N, _ = Y.shape
    _, C, F, _ = x.shape
    M = C * F
    x2 = x.reshape(B, M, N)
    TN = 1024
    out = pl.pallas_call(
        _mm_kernel,
        grid=(B, N // TN),
        in_specs=[
            pl.BlockSpec((1, M, N), lambda b, j: (b, 0, 0)),
            pl.BlockSpec((1, N, TN), lambda b, j: (b, 0, j)),
        ],
        out_specs=pl.BlockSpec((1, M, TN), lambda b, j: (b, 0, j)),
        out_shape=jax.ShapeDtypeStruct((B, M, N), jnp.float32),
    )(x2, Y)
    return out.reshape(B, C, F, N)
